# baseline (speedup 1.0000x reference)
import functools

import jax
import jax.numpy as jnp
from jax import lax
from jax.experimental import pallas as pl
from jax.experimental.pallas import tpu as pltpu
from jax.experimental.pallas import tpu_sc as plsc

_L = 50
_LP = 56
_HID = 128
_DIM = 2 * _HID
_NC = 2
_NS = 16


def _sc_gather(idx2d, emb):
    nw = _NC * _NS
    n_sessions = idx2d.shape[0]
    total_rows = n_sessions * _LP
    rows_per_w = total_rows // nw
    chunk = 112
    n_chunks = rows_per_w // chunk
    h = emb.shape[1]
    idx_rows = idx2d.reshape(-1, chunk)
    mesh = plsc.VectorSubcoreMesh(core_axis_name="c", subcore_axis_name="s")

    @functools.partial(
        pl.kernel,
        mesh=mesh,
        out_type=jax.ShapeDtypeStruct((total_rows, h), jnp.float32),
        scratch_types=[
            pltpu.VMEM((n_chunks, chunk), jnp.int32),
            pltpu.VMEM((rows_per_w, h), jnp.float32),
            pltpu.SemaphoreType.DMA,
            pltpu.SemaphoreType.DMA,
        ],
    )
    def gather_kernel(idx_hbm, emb_hbm, out_hbm, idx_v, rows_v, gsem, wsem):
        wid = lax.axis_index("s") * _NC + lax.axis_index("c")
        pltpu.sync_copy(idx_hbm.at[pl.ds(wid * n_chunks, n_chunks)], idx_v)
        copies = [
            pltpu.async_copy(
                emb_hbm.at[idx_v.at[c]],
                rows_v.at[pl.ds(c * chunk, chunk)],
                gsem,
            )
            for c in range(n_chunks)
        ]
        writes = []
        for c in range(n_chunks):
            copies[c].wait()
            writes.append(pltpu.async_copy(
                rows_v.at[pl.ds(c * chunk, chunk)],
                out_hbm.at[pl.ds(wid * rows_per_w + c * chunk, chunk)],
                wsem,
            ))
        for wr in writes:
            wr.wait()

    return gather_kernel(idx_rows, emb)


def _tc_gnn(h_pad, pos_p, a_in, a_out, win_t, wout_t, wih_t, whh_t,
            b_in, b_out, b_ih, b_hh, b_iah, b_oah, bb):
    n_sessions = a_in.shape[0]
    r = bb * _LP

    def body(h_ref, p_ref, ain_ref, aout_ref, win_ref, wout_ref, wih_ref,
             whh_ref, bin_ref, bout_ref, bih_ref, bhh_ref, biah_ref,
             boah_ref, out_ref, scr):
        x = h_ref[...]
        nrm = jnp.sqrt(jnp.sum(x * x, axis=1, keepdims=True))
        x = x * jnp.minimum(1.0, 1.5 / (nrm + 1e-7))
        row = lax.broadcasted_iota(jnp.int32, (r, 1), 0)
        x = jnp.where((row % _LP) < _L, x, 0.0)
        p = p_ref[...]
        pn = jnp.sqrt(jnp.sum(p * p, axis=1, keepdims=True))
        p = p * jnp.minimum(1.0, 1.5 / (pn + 1e-7))
        p_t = jnp.concatenate([p] * bb, axis=0)
        hidden = jnp.concatenate([x, p_t], axis=1)

        hb = hidden.astype(jnp.bfloat16)
        ein = jnp.dot(hb, win_ref[...],
                      preferred_element_type=jnp.float32) + bin_ref[...]
        eout = jnp.dot(hb, wout_ref[...],
                       preferred_element_type=jnp.float32) + bout_ref[...]
        for s in range(bb):
            sl = pl.ds(s * _LP, _LP)
            scr[sl, 0:_DIM] = jnp.dot(
                ain_ref[s], ein[s * _LP:(s + 1) * _LP].astype(jnp.bfloat16),
                preferred_element_type=jnp.float32) + biah_ref[...]
            scr[sl, _DIM:2 * _DIM] = jnp.dot(
                aout_ref[s], eout[s * _LP:(s + 1) * _LP].astype(jnp.bfloat16),
                preferred_element_type=jnp.float32) + boah_ref[...]
        gi = jnp.dot(scr[...].astype(jnp.bfloat16), wih_ref[...],
                     preferred_element_type=jnp.float32) + bih_ref[...]
        gh = jnp.dot(hb, whh_ref[...],
                     preferred_element_type=jnp.float32) + bhh_ref[...]
        rg = jax.nn.sigmoid(gi[:, :_DIM] + gh[:, :_DIM])
        ig = jax.nn.sigmoid(gi[:, _DIM:2 * _DIM] + gh[:, _DIM:2 * _DIM])
        ng = jnp.tanh(gi[:, 2 * _DIM:] + rg * gh[:, 2 * _DIM:])
        hnew = ng + ig * (hidden - ng)
        for s in range(bb):
            out_ref[s] = hnew[s * _LP:s * _LP + _L, :]

    return pl.pallas_call(
        body,
        grid=(n_sessions // bb,),
        in_specs=[
            pl.BlockSpec((r, _HID), lambda i: (i, 0)),
            pl.BlockSpec((_LP, _HID), lambda i: (0, 0)),
            pl.BlockSpec((bb, _LP, _LP), lambda i: (i, 0, 0)),
            pl.BlockSpec((bb, _LP, _LP), lambda i: (i, 0, 0)),
            pl.BlockSpec((_DIM, _DIM), lambda i: (0, 0)),
            pl.BlockSpec((_DIM, _DIM), lambda i: (0, 0)),
            pl.BlockSpec((2 * _DIM, 3 * _DIM), lambda i: (0, 0)),
            pl.BlockSpec((_DIM, 3 * _DIM), lambda i: (0, 0)),
            pl.BlockSpec((1, _DIM), lambda i: (0, 0)),
            pl.BlockSpec((1, _DIM), lambda i: (0, 0)),
            pl.BlockSpec((1, 3 * _DIM), lambda i: (0, 0)),
            pl.BlockSpec((1, 3 * _DIM), lambda i: (0, 0)),
            pl.BlockSpec((1, _DIM), lambda i: (0, 0)),
            pl.BlockSpec((1, _DIM), lambda i: (0, 0)),
        ],
        out_specs=pl.BlockSpec((bb, _L, _DIM), lambda i: (i, 0, 0)),
        out_shape=jax.ShapeDtypeStruct((n_sessions, _L, _DIM), jnp.float32),
        scratch_shapes=[pltpu.VMEM((r, 2 * _DIM), jnp.float32)],
        compiler_params=pltpu.CompilerParams(
            dimension_semantics=("parallel",)),
    )(h_pad, pos_p, a_in, a_out, win_t, wout_t, wih_t, whh_t,
      b_in, b_out, b_ih, b_hh, b_iah, b_oah)


def kernel(items, A, emb, pos, W_in, b_in, W_out, b_out,
           w_ih, w_hh, b_ih, b_hh, b_iah, b_oah):
    n_sessions, seq_len = items.shape
    pad = _LP - _L
    idx2d = jnp.pad(items.astype(jnp.int32), ((0, 0), (0, pad)), mode="wrap")
    h_pad = _sc_gather(idx2d, emb)
    bf = jnp.bfloat16
    a_in = jnp.pad(A[:, :, :seq_len], ((0, 0), (0, pad), (0, pad))).astype(bf)
    a_out = jnp.pad(A[:, :, seq_len:], ((0, 0), (0, pad), (0, pad))).astype(bf)
    pos_p = pos[:_LP]

    bb = 8
    out = _tc_gnn(
        h_pad, pos_p, a_in, a_out,
        W_in.T.astype(bf), W_out.T.astype(bf),
        w_ih.T.astype(bf), w_hh.T.astype(bf),
        b_in.reshape(1, -1), b_out.reshape(1, -1),
        b_ih.reshape(1, -1), b_hh.reshape(1, -1),
        b_iah.reshape(1, -1), b_oah.reshape(1, -1), bb)
    return out


# speedup vs baseline: 1.0854x; 1.0854x over previous
import functools

import jax
import jax.numpy as jnp
from jax import lax
from jax.experimental import pallas as pl
from jax.experimental.pallas import tpu as pltpu
from jax.experimental.pallas import tpu_sc as plsc

_L = 50
_LP = 56
_HID = 128
_DIM = 2 * _HID
_NC = 2
_NS = 16


def _sc_gather(idx2d, emb):
    nw = _NC * _NS
    n_sessions = idx2d.shape[0]
    total_rows = n_sessions * _LP
    rows_per_w = total_rows // nw
    chunk = 112
    n_chunks = rows_per_w // chunk
    h = emb.shape[1]
    idx_rows = idx2d.reshape(-1, chunk)
    mesh = plsc.VectorSubcoreMesh(core_axis_name="c", subcore_axis_name="s")

    @functools.partial(
        pl.kernel,
        mesh=mesh,
        out_type=jax.ShapeDtypeStruct((total_rows, h), jnp.float32),
        scratch_types=[
            pltpu.VMEM((n_chunks, chunk), jnp.int32),
            pltpu.VMEM((rows_per_w, h), jnp.float32),
            pltpu.SemaphoreType.DMA,
            pltpu.SemaphoreType.DMA,
        ],
    )
    def gather_kernel(idx_hbm, emb_hbm, out_hbm, idx_v, rows_v, gsem, wsem):
        wid = lax.axis_index("s") * _NC + lax.axis_index("c")
        pltpu.sync_copy(idx_hbm.at[pl.ds(wid * n_chunks, n_chunks)], idx_v)
        copies = [
            pltpu.async_copy(
                emb_hbm.at[idx_v.at[c]],
                rows_v.at[pl.ds(c * chunk, chunk)],
                gsem,
            )
            for c in range(n_chunks)
        ]
        writes = []
        for c in range(n_chunks):
            copies[c].wait()
            writes.append(pltpu.async_copy(
                rows_v.at[pl.ds(c * chunk, chunk)],
                out_hbm.at[pl.ds(wid * rows_per_w + c * chunk, chunk)],
                wsem,
            ))
        for wr in writes:
            wr.wait()

    return gather_kernel(idx_rows, emb)


def _tc_gnn(h_pad, pos_p, a_in, a_out, win_t, wout_t, wih_t, whh_t,
            b_in, b_out, b_ih, b_hh, b_iah, b_oah, bb):
    n_sessions = a_in.shape[0]
    r = bb * _LP

    def body(h_ref, p_ref, ain_ref, aout_ref, win_ref, wout_ref, wih_ref,
             whh_ref, bin_ref, bout_ref, bih_ref, bhh_ref, biah_ref,
             boah_ref, out_ref, scr):
        x = h_ref[...]
        nrm = jnp.sqrt(jnp.sum(x * x, axis=1, keepdims=True))
        x = x * jnp.minimum(1.0, 1.5 / (nrm + 1e-7))
        row = lax.broadcasted_iota(jnp.int32, (r, 1), 0)
        x = jnp.where((row % _LP) < _L, x, 0.0)
        p = p_ref[...]
        pn = jnp.sqrt(jnp.sum(p * p, axis=1, keepdims=True))
        p = p * jnp.minimum(1.0, 1.5 / (pn + 1e-7))
        p_t = jnp.concatenate([p] * bb, axis=0)
        hidden = jnp.concatenate([x, p_t], axis=1)

        xb = x.astype(jnp.bfloat16)
        pb = p.astype(jnp.bfloat16)

        def _xp_dot(w_ref, bias):
            px = jnp.dot(pb, w_ref[_HID:, :],
                         preferred_element_type=jnp.float32) + bias
            return (jnp.dot(xb, w_ref[:_HID, :],
                            preferred_element_type=jnp.float32)
                    + jnp.concatenate([px] * bb, axis=0))

        ein = _xp_dot(win_ref, bin_ref[...])
        eout = _xp_dot(wout_ref, bout_ref[...])
        for s in range(bb):
            sl = pl.ds(s * _LP, _LP)
            scr[sl, 0:_DIM] = jnp.dot(
                ain_ref[s], ein[s * _LP:(s + 1) * _LP].astype(jnp.bfloat16),
                preferred_element_type=jnp.float32) + biah_ref[...]
            scr[sl, _DIM:2 * _DIM] = jnp.dot(
                aout_ref[s], eout[s * _LP:(s + 1) * _LP].astype(jnp.bfloat16),
                preferred_element_type=jnp.float32) + boah_ref[...]
        gi = jnp.dot(scr[...].astype(jnp.bfloat16), wih_ref[...],
                     preferred_element_type=jnp.float32) + bih_ref[...]
        gh = _xp_dot(whh_ref, bhh_ref[...])
        rg = jax.nn.sigmoid(gi[:, :_DIM] + gh[:, :_DIM])
        ig = jax.nn.sigmoid(gi[:, _DIM:2 * _DIM] + gh[:, _DIM:2 * _DIM])
        ng = jnp.tanh(gi[:, 2 * _DIM:] + rg * gh[:, 2 * _DIM:])
        hnew = ng + ig * (hidden - ng)
        for s in range(bb):
            out_ref[s] = hnew[s * _LP:s * _LP + _L, :]

    return pl.pallas_call(
        body,
        grid=(n_sessions // bb,),
        in_specs=[
            pl.BlockSpec((r, _HID), lambda i: (i, 0)),
            pl.BlockSpec((_LP, _HID), lambda i: (0, 0)),
            pl.BlockSpec((bb, _LP, _LP), lambda i: (i, 0, 0)),
            pl.BlockSpec((bb, _LP, _LP), lambda i: (i, 0, 0)),
            pl.BlockSpec((_DIM, _DIM), lambda i: (0, 0)),
            pl.BlockSpec((_DIM, _DIM), lambda i: (0, 0)),
            pl.BlockSpec((2 * _DIM, 3 * _DIM), lambda i: (0, 0)),
            pl.BlockSpec((_DIM, 3 * _DIM), lambda i: (0, 0)),
            pl.BlockSpec((1, _DIM), lambda i: (0, 0)),
            pl.BlockSpec((1, _DIM), lambda i: (0, 0)),
            pl.BlockSpec((1, 3 * _DIM), lambda i: (0, 0)),
            pl.BlockSpec((1, 3 * _DIM), lambda i: (0, 0)),
            pl.BlockSpec((1, _DIM), lambda i: (0, 0)),
            pl.BlockSpec((1, _DIM), lambda i: (0, 0)),
        ],
        out_specs=pl.BlockSpec((bb, _L, _DIM), lambda i: (i, 0, 0)),
        out_shape=jax.ShapeDtypeStruct((n_sessions, _L, _DIM), jnp.float32),
        scratch_shapes=[pltpu.VMEM((r, 2 * _DIM), jnp.float32)],
        compiler_params=pltpu.CompilerParams(
            dimension_semantics=("parallel",)),
    )(h_pad, pos_p, a_in, a_out, win_t, wout_t, wih_t, whh_t,
      b_in, b_out, b_ih, b_hh, b_iah, b_oah)


def kernel(items, A, emb, pos, W_in, b_in, W_out, b_out,
           w_ih, w_hh, b_ih, b_hh, b_iah, b_oah):
    n_sessions, seq_len = items.shape
    pad = _LP - _L
    idx2d = jnp.pad(items.astype(jnp.int32), ((0, 0), (0, pad)), mode="wrap")
    h_pad = _sc_gather(idx2d, emb)
    bf = jnp.bfloat16
    a_in = jnp.pad(A[:, :, :seq_len], ((0, 0), (0, pad), (0, pad))).astype(bf)
    a_out = jnp.pad(A[:, :, seq_len:], ((0, 0), (0, pad), (0, pad))).astype(bf)
    pos_p = pos[:_LP]

    bb = 32
    out = _tc_gnn(
        h_pad, pos_p, a_in, a_out,
        W_in.T.astype(bf), W_out.T.astype(bf),
        w_ih.T.astype(bf), w_hh.T.astype(bf),
        b_in.reshape(1, -1), b_out.reshape(1, -1),
        b_ih.reshape(1, -1), b_hh.reshape(1, -1),
        b_iah.reshape(1, -1), b_oah.reshape(1, -1), bb)
    return out
